# baseline (speedup 1.0000x reference)
import functools

import jax
import jax.numpy as jnp
from jax import lax
from jax.experimental import pallas as pl
from jax.experimental.pallas import tpu as pltpu
from jax.experimental.pallas import tpu_sc as plsc

N_NODES = 10000
N_EDGES = 320000
D_EDGE = 16
D_FEAT = 128
D_GLOBAL = 32
H_DIM = 128

GRP = 128
NG = N_EDGES // GRP
NC = 2
NS = 16
GRP_PER_SC = NG // NC
GRP_BASE = GRP_PER_SC // NS
GRP_EXTRA = GRP_PER_SC - GRP_BASE * NS
W_GRPS = 13
N_WIN = GRP_BASE // W_GRPS
ROWS_PER_TILE = N_NODES // NS


def _sc_segment_sum(edges4d, idx2d):
  mesh = plsc.VectorSubcoreMesh(core_axis_name="c", subcore_axis_name="s")

  @functools.partial(
      pl.kernel,
      out_type=jax.ShapeDtypeStruct((NC, N_NODES, D_EDGE), jnp.float32),
      mesh=mesh,
      scratch_types=[
          pltpu.VMEM((W_GRPS, GRP), jnp.int32),
          pltpu.VMEM((2, W_GRPS, 8, GRP), jnp.float32),
          pltpu.VMEM((W_GRPS * GRP, D_EDGE), jnp.float32),
          pltpu.VMEM((ROWS_PER_TILE, D_EDGE), jnp.float32),
          pltpu.VMEM_SHARED((N_NODES, D_EDGE), jnp.float32),
          pltpu.SemaphoreType.DMA,
      ],
      compiler_params=pltpu.CompilerParams(use_tc_tiling_on_sc=False,
                                           needs_layout_passes=False),
  )
  def seg_sum(edges_hbm, idx_hbm, out_hbm, idx_v, data_v, rows_v, buf_v,
              acc_sh, sem):
    c = lax.axis_index("c")
    s = lax.axis_index("s")

    zrow = jnp.zeros((D_EDGE,), jnp.float32)

    def zero_body(i, carry):
      buf_v[i] = zrow
      return carry

    lax.fori_loop(0, ROWS_PER_TILE, zero_body, 0)
    pltpu.sync_copy(buf_v, acc_sh.at[pl.ds(s * ROWS_PER_TILE, ROWS_PER_TILE)])
    plsc.subcore_barrier()

    base = c * GRP_PER_SC + s * GRP_BASE + jnp.minimum(s, GRP_EXTRA)

    lane = lax.iota(jnp.int32, 16)
    fbv = lane >> 3
    fiv = lane & 7

    def do_groups(n_groups, g0):
      pltpu.sync_copy(idx_hbm.at[pl.ds(g0, n_groups)],
                      idx_v.at[pl.ds(0, n_groups)])
      pltpu.sync_copy(edges_hbm.at[:, pl.ds(g0, n_groups)],
                      data_v.at[:, pl.ds(0, n_groups)])
      scatters = []
      for g in range(n_groups):
        gv = jnp.full((16,), g, jnp.int32)

        def blk16(b, carry):
          for k in range(16):
            e = b * 16 + k
            ev = jnp.full((16,), e, jnp.int32)
            row = plsc.load_gather(data_v, [fbv, gv, fiv, ev])
            rows_v[g * GRP + e] = row
          return carry

        lax.fori_loop(0, GRP // 16, blk16, 0)
        scatters.append(
            pltpu.async_copy(rows_v.at[pl.ds(g * GRP, GRP)],
                             acc_sh.at[idx_v.at[g]], sem, add=True))
      for d in scatters:
        d.wait()

    def window(w, carry):
      do_groups(W_GRPS, base + w * W_GRPS)
      return carry

    lax.fori_loop(0, N_WIN, window, 0)

    @pl.when(s < GRP_EXTRA)
    def _extra():
      do_groups(1, base + GRP_BASE)

    plsc.subcore_barrier()

    pltpu.sync_copy(acc_sh.at[pl.ds(s * ROWS_PER_TILE, ROWS_PER_TILE)], buf_v)
    pltpu.sync_copy(buf_v,
                    out_hbm.at[c].at[pl.ds(s * ROWS_PER_TILE, ROWS_PER_TILE)])

  return seg_sum(edges4d, idx2d)


ROW_BLK = 1000


def _tc_mlp_ln(nodes, agg2, globals_, W1, b1, W2, b2, gamma, beta):
  grid = (N_NODES // ROW_BLK,)

  def body(nodes_ref, agg_ref, g_ref, w1_ref, b1_ref, w2_ref, b2_ref,
           gamma_ref, beta_ref, out_ref):
    agg = agg_ref[0] + agg_ref[1]
    w1n = w1_ref[:D_FEAT]
    w1g = w1_ref[D_FEAT:D_FEAT + D_GLOBAL]
    w1f = w1_ref[D_FEAT + D_GLOBAL:]
    bias1 = b1_ref[...] + jnp.dot(g_ref[...], w1g,
                                  preferred_element_type=jnp.float32)
    x = (jnp.dot(nodes_ref[...], w1n, preferred_element_type=jnp.float32)
         + jnp.dot(agg, w1f, preferred_element_type=jnp.float32)
         + bias1)
    h = jnp.maximum(x, 0.0)
    h = jnp.dot(h, w2_ref[...], preferred_element_type=jnp.float32)
    h = jnp.maximum(h + b2_ref[...], 0.0)
    mean = jnp.mean(h, axis=1, keepdims=True)
    d = h - mean
    var = jnp.mean(d * d, axis=1, keepdims=True)
    out_ref[...] = gamma_ref[...] * d * lax.rsqrt(var + 1e-3) + beta_ref[...]

  return pl.pallas_call(
      body,
      grid=grid,
      in_specs=[
          pl.BlockSpec((ROW_BLK, D_FEAT), lambda i: (i, 0)),
          pl.BlockSpec((NC, ROW_BLK, D_EDGE), lambda i: (0, i, 0)),
          pl.BlockSpec((1, D_GLOBAL), lambda i: (0, 0)),
          pl.BlockSpec((D_FEAT + D_GLOBAL + D_EDGE, H_DIM), lambda i: (0, 0)),
          pl.BlockSpec((1, H_DIM), lambda i: (0, 0)),
          pl.BlockSpec((H_DIM, H_DIM), lambda i: (0, 0)),
          pl.BlockSpec((1, H_DIM), lambda i: (0, 0)),
          pl.BlockSpec((1, H_DIM), lambda i: (0, 0)),
          pl.BlockSpec((1, H_DIM), lambda i: (0, 0)),
      ],
      out_specs=pl.BlockSpec((ROW_BLK, H_DIM), lambda i: (i, 0)),
      out_shape=jax.ShapeDtypeStruct((N_NODES, H_DIM), jnp.float32),
      compiler_params=pltpu.CompilerParams(
          dimension_semantics=("arbitrary",),
      ),
  )(nodes, agg2, globals_, W1, b1, W2, b2, gamma, beta)


@jax.jit
def kernel(nodes, globals_, n_node, hyperedges, hyperedge_index,
           W1, b1, W2, b2, gamma, beta):
  del n_node
  idx2d = hyperedge_index.reshape(NG, GRP)
  edges4d = hyperedges.T.reshape(2, 8, NG, GRP).transpose(0, 2, 1, 3)
  agg2 = _sc_segment_sum(edges4d, idx2d)
  return _tc_mlp_ln(
      nodes, agg2, globals_, W1,
      b1.reshape(1, H_DIM), W2, b2.reshape(1, H_DIM),
      gamma.reshape(1, H_DIM), beta.reshape(1, H_DIM),
  )


# speedup vs baseline: 1.1292x; 1.1292x over previous
import functools

import jax
import jax.numpy as jnp
from jax import lax
from jax.experimental import pallas as pl
from jax.experimental.pallas import tpu as pltpu
from jax.experimental.pallas import tpu_sc as plsc

N_NODES = 10000
N_EDGES = 320000
D_EDGE = 16
D_FEAT = 128
D_GLOBAL = 32
H_DIM = 128

GRP = 128
NG = N_EDGES // GRP
NC = 2
NS = 16
GRP_PER_SC = NG // NC
GRP_BASE = GRP_PER_SC // NS
GRP_EXTRA = GRP_PER_SC - GRP_BASE * NS
W_GRPS = 13
N_WIN = GRP_BASE // W_GRPS
ROWS_PER_TILE = N_NODES // NS


WIN_WORDS = 2 * W_GRPS * 8 * GRP
HALF_WIN = W_GRPS * 8 * GRP


def _sc_segment_sum(edges1d, idx2d):
  mesh = plsc.VectorSubcoreMesh(core_axis_name="c", subcore_axis_name="s")

  @functools.partial(
      pl.kernel,
      out_type=jax.ShapeDtypeStruct((NC, N_NODES, D_EDGE), jnp.float32),
      mesh=mesh,
      scratch_types=[
          pltpu.VMEM((N_WIN, W_GRPS, GRP), jnp.int32),
          pltpu.VMEM((2, WIN_WORDS), jnp.float32),
          pltpu.VMEM((2, W_GRPS * GRP, D_EDGE), jnp.float32),
          pltpu.VMEM_SHARED((N_NODES, D_EDGE), jnp.float32),
          pltpu.SemaphoreType.DMA,
          pltpu.SemaphoreType.DMA,
          pltpu.SemaphoreType.DMA,
          pltpu.SemaphoreType.DMA,
      ],
      compiler_params=pltpu.CompilerParams(use_tc_tiling_on_sc=False,
                                           needs_layout_passes=False),
  )
  def seg_sum(edges_hbm, idx_hbm, out_hbm, idx_v, data_v, rows_v, acc_sh,
              si0, si1, ss0, ss1):
    c = lax.axis_index("c")
    s = lax.axis_index("s")

    zrow = jnp.zeros((D_EDGE,), jnp.float32)

    def zero_body(i, carry):
      rows_v[0, i] = zrow
      return carry

    lax.fori_loop(0, ROWS_PER_TILE, zero_body, 0)
    pltpu.sync_copy(rows_v.at[0, pl.ds(0, ROWS_PER_TILE)],
                    acc_sh.at[pl.ds(s * ROWS_PER_TILE, ROWS_PER_TILE)])
    plsc.subcore_barrier()

    base = c * GRP_PER_SC + s * GRP_BASE + jnp.minimum(s, GRP_EXTRA)

    lane = lax.iota(jnp.int32, 16)
    fbase = (lane >> 3) * HALF_WIN + (lane & 7) * GRP

    def start_in(w, b, sem):
      g0 = base + w * W_GRPS
      pltpu.async_copy(idx_hbm.at[pl.ds(g0, W_GRPS)], idx_v.at[w], sem)
      for fb in range(2):
        pltpu.async_copy(
            edges_hbm.at[pl.ds(fb * (8 * NG * GRP) + g0 * (8 * GRP),
                               HALF_WIN)],
            data_v.at[b, pl.ds(fb * HALF_WIN, HALF_WIN)], sem)

    def drain_in(b, sem):
      pltpu.make_async_copy(idx_hbm.at[pl.ds(0, W_GRPS)], idx_v.at[0],
                            sem).wait()
      pltpu.make_async_copy(edges_hbm.at[pl.ds(0, WIN_WORDS)], data_v.at[b],
                            sem).wait()

    def drain_sc(b, sem):
      pltpu.make_async_copy(out_hbm.at[0].at[pl.ds(0, W_GRPS * GRP)],
                            rows_v.at[b], sem).wait()

    def transpose_scatter(w, b, sem):
      def grp_body(g, carry):
        gbase = fbase + g * 1024

        def blk16(k16, carry2):
          e0 = k16 * 16
          for k in range(16):
            row = plsc.load_gather(data_v.at[b], [gbase + (e0 + k)])
            rows_v[b, g * GRP + e0 + k] = row
          return carry2

        lax.fori_loop(0, GRP // 16, blk16, 0)
        pltpu.async_copy(rows_v.at[b, pl.ds(g * GRP, GRP)],
                         acc_sh.at[idx_v.at[w].at[g]], sem, add=True)
        return carry

      lax.fori_loop(0, W_GRPS, grp_body, 0)

    start_in(0, 0, si0)
    start_in(1, 1, si1)

    def tbody(t, carry):
      w0 = 2 * t
      drain_in(0, si0)

      @pl.when(t >= 1)
      def _():
        drain_sc(0, ss0)

      transpose_scatter(w0, 0, ss0)

      @pl.when(w0 + 2 < N_WIN)
      def _():
        start_in(w0 + 2, 0, si0)

      drain_in(1, si1)

      @pl.when(t >= 1)
      def _():
        drain_sc(1, ss1)

      transpose_scatter(w0 + 1, 1, ss1)

      @pl.when(w0 + 3 < N_WIN)
      def _():
        start_in(w0 + 3, 1, si1)

      return carry

    lax.fori_loop(0, N_WIN // 2, tbody, 0)
    drain_sc(0, ss0)
    drain_sc(1, ss1)

    @pl.when(s < GRP_EXTRA)
    def _extra():
      g0 = base + GRP_BASE
      pltpu.sync_copy(idx_hbm.at[pl.ds(g0, 1)], idx_v.at[0, pl.ds(0, 1)])
      for fb in range(2):
        pltpu.sync_copy(
            edges_hbm.at[pl.ds(fb * (8 * NG * GRP) + g0 * (8 * GRP),
                               8 * GRP)],
            data_v.at[0, pl.ds(fb * HALF_WIN, 8 * GRP)])

      def blk16(k16, carry):
        e0 = k16 * 16
        for k in range(16):
          row = plsc.load_gather(data_v.at[0], [fbase + (e0 + k)])
          rows_v[0, e0 + k] = row
        return carry

      lax.fori_loop(0, GRP // 16, blk16, 0)
      pltpu.sync_copy(rows_v.at[0, pl.ds(0, GRP)],
                      acc_sh.at[idx_v.at[0].at[0]], add=True)

    plsc.subcore_barrier()

    pltpu.sync_copy(acc_sh.at[pl.ds(s * ROWS_PER_TILE, ROWS_PER_TILE)],
                    rows_v.at[0, pl.ds(0, ROWS_PER_TILE)])
    pltpu.sync_copy(rows_v.at[0, pl.ds(0, ROWS_PER_TILE)],
                    out_hbm.at[c].at[pl.ds(s * ROWS_PER_TILE, ROWS_PER_TILE)])

  return seg_sum(edges1d, idx2d)


ROW_BLK = 1000


def _tc_mlp_ln(nodes, agg2, globals_, W1, b1, W2, b2, gamma, beta):
  grid = (N_NODES // ROW_BLK,)

  def body(nodes_ref, agg_ref, g_ref, w1_ref, b1_ref, w2_ref, b2_ref,
           gamma_ref, beta_ref, out_ref):
    agg = agg_ref[0] + agg_ref[1]
    w1n = w1_ref[:D_FEAT]
    w1g = w1_ref[D_FEAT:D_FEAT + D_GLOBAL]
    w1f = w1_ref[D_FEAT + D_GLOBAL:]
    bias1 = b1_ref[...] + jnp.dot(g_ref[...], w1g,
                                  preferred_element_type=jnp.float32)
    x = (jnp.dot(nodes_ref[...], w1n, preferred_element_type=jnp.float32)
         + jnp.dot(agg, w1f, preferred_element_type=jnp.float32)
         + bias1)
    h = jnp.maximum(x, 0.0)
    h = jnp.dot(h, w2_ref[...], preferred_element_type=jnp.float32)
    h = jnp.maximum(h + b2_ref[...], 0.0)
    mean = jnp.mean(h, axis=1, keepdims=True)
    d = h - mean
    var = jnp.mean(d * d, axis=1, keepdims=True)
    out_ref[...] = gamma_ref[...] * d * lax.rsqrt(var + 1e-3) + beta_ref[...]

  return pl.pallas_call(
      body,
      grid=grid,
      in_specs=[
          pl.BlockSpec((ROW_BLK, D_FEAT), lambda i: (i, 0)),
          pl.BlockSpec((NC, ROW_BLK, D_EDGE), lambda i: (0, i, 0)),
          pl.BlockSpec((1, D_GLOBAL), lambda i: (0, 0)),
          pl.BlockSpec((D_FEAT + D_GLOBAL + D_EDGE, H_DIM), lambda i: (0, 0)),
          pl.BlockSpec((1, H_DIM), lambda i: (0, 0)),
          pl.BlockSpec((H_DIM, H_DIM), lambda i: (0, 0)),
          pl.BlockSpec((1, H_DIM), lambda i: (0, 0)),
          pl.BlockSpec((1, H_DIM), lambda i: (0, 0)),
          pl.BlockSpec((1, H_DIM), lambda i: (0, 0)),
      ],
      out_specs=pl.BlockSpec((ROW_BLK, H_DIM), lambda i: (i, 0)),
      out_shape=jax.ShapeDtypeStruct((N_NODES, H_DIM), jnp.float32),
      compiler_params=pltpu.CompilerParams(
          dimension_semantics=("arbitrary",),
      ),
  )(nodes, agg2, globals_, W1, b1, W2, b2, gamma, beta)


@jax.jit
def kernel(nodes, globals_, n_node, hyperedges, hyperedge_index,
           W1, b1, W2, b2, gamma, beta):
  del n_node
  idx2d = hyperedge_index.reshape(NG, GRP)
  edges1d = (hyperedges.T.reshape(2, 8, NG, GRP)
             .transpose(0, 2, 1, 3).reshape(-1))
  agg2 = _sc_segment_sum(edges1d, idx2d)
  return _tc_mlp_ln(
      nodes, agg2, globals_, W1,
      b1.reshape(1, H_DIM), W2, b2.reshape(1, H_DIM),
      gamma.reshape(1, H_DIM), beta.reshape(1, H_DIM),
  )


# speedup vs baseline: 1.8054x; 1.5989x over previous
import functools

import jax
import jax.numpy as jnp
from jax import lax
from jax.experimental import pallas as pl
from jax.experimental.pallas import tpu as pltpu
from jax.experimental.pallas import tpu_sc as plsc

N_NODES = 10000
N_EDGES = 320000
D_EDGE = 16
D_FEAT = 128
D_GLOBAL = 32
H_DIM = 128

GRP = 128
NG = N_EDGES // GRP
NC = 2
NS = 16
GRP_PER_SC = NG // NC
GRP_BASE = GRP_PER_SC // NS
GRP_EXTRA = GRP_PER_SC - GRP_BASE * NS
W_GRPS = 13
N_WIN = GRP_BASE // W_GRPS
ROWS_PER_TILE = N_NODES // NS


WIN_ROWS = 2 * W_GRPS * 8
PAD = GRP + 1


def _sc_segment_sum(edges2d, idx2d):
  mesh = plsc.VectorSubcoreMesh(core_axis_name="c", subcore_axis_name="s")

  @functools.partial(
      pl.kernel,
      out_type=jax.ShapeDtypeStruct((NC, N_NODES, D_EDGE), jnp.float32),
      mesh=mesh,
      scratch_types=[
          pltpu.VMEM((N_WIN, W_GRPS, GRP), jnp.int32),
          pltpu.VMEM((2, WIN_ROWS, PAD), jnp.float32),
          pltpu.VMEM((2, W_GRPS * GRP, D_EDGE), jnp.float32),
          pltpu.VMEM_SHARED((N_NODES, D_EDGE), jnp.float32),
          pltpu.SemaphoreType.DMA,
          pltpu.SemaphoreType.DMA,
          pltpu.SemaphoreType.DMA,
          pltpu.SemaphoreType.DMA,
      ],
      compiler_params=pltpu.CompilerParams(use_tc_tiling_on_sc=False,
                                           needs_layout_passes=False),
  )
  def seg_sum(edges_hbm, idx_hbm, out_hbm, idx_v, data_v, rows_v, acc_sh,
              si0, si1, ss0, ss1):
    c = lax.axis_index("c")
    s = lax.axis_index("s")

    zrow = jnp.zeros((D_EDGE,), jnp.float32)

    def zero_body(i, carry):
      rows_v[0, i] = zrow
      return carry

    lax.fori_loop(0, ROWS_PER_TILE, zero_body, 0)
    pltpu.sync_copy(rows_v.at[0, pl.ds(0, ROWS_PER_TILE)],
                    acc_sh.at[pl.ds(s * ROWS_PER_TILE, ROWS_PER_TILE)])
    plsc.subcore_barrier()

    base = c * GRP_PER_SC + s * GRP_BASE + jnp.minimum(s, GRP_EXTRA)

    lane = lax.iota(jnp.int32, 16)
    frow = (lane >> 3) * (8 * W_GRPS) + (lane & 7)

    def start_in(w, b, sem):
      g0 = base + w * W_GRPS
      pltpu.async_copy(idx_hbm.at[pl.ds(g0, W_GRPS)], idx_v.at[w], sem)
      for fb in range(2):
        pltpu.async_copy(
            edges_hbm.at[pl.ds(fb * (8 * NG) + g0 * 8, 8 * W_GRPS)],
            data_v.at[b, pl.ds(fb * 8 * W_GRPS, 8 * W_GRPS),
                      pl.ds(0, GRP)], sem)

    def drain_in(b, sem):
      del b
      pltpu.make_async_copy(idx_hbm.at[pl.ds(0, W_GRPS)], idx_v.at[0],
                            sem).wait()
      pltpu.make_async_copy(out_hbm.at[0].at[pl.ds(0, W_GRPS * GRP)],
                            rows_v.at[0], sem).wait()

    def drain_sc(b, sem):
      pltpu.make_async_copy(out_hbm.at[0].at[pl.ds(0, W_GRPS * GRP)],
                            rows_v.at[b], sem).wait()

    def transpose_scatter(w, b, sem):
      def grp_body(g, carry):
        rowg = frow + g * 8

        def blk16(k16, carry2):
          e0 = k16 * 16
          for k in range(16):
            ev = jnp.full((16,), e0 + k, jnp.int32)
            row = plsc.load_gather(data_v.at[b], [rowg, ev])
            rows_v[b, g * GRP + e0 + k] = row
          return carry2

        lax.fori_loop(0, GRP // 16, blk16, 0)
        pltpu.async_copy(rows_v.at[b, pl.ds(g * GRP, GRP)],
                         acc_sh.at[idx_v.at[w].at[g]], sem, add=True)
        return carry

      lax.fori_loop(0, W_GRPS, grp_body, 0)

    start_in(0, 0, si0)
    start_in(1, 1, si1)

    def tbody(t, carry):
      w0 = 2 * t
      drain_in(0, si0)

      @pl.when(t >= 1)
      def _():
        drain_sc(0, ss0)

      transpose_scatter(w0, 0, ss0)

      @pl.when(w0 + 2 < N_WIN)
      def _():
        start_in(w0 + 2, 0, si0)

      drain_in(1, si1)

      @pl.when(t >= 1)
      def _():
        drain_sc(1, ss1)

      transpose_scatter(w0 + 1, 1, ss1)

      @pl.when(w0 + 3 < N_WIN)
      def _():
        start_in(w0 + 3, 1, si1)

      return carry

    lax.fori_loop(0, N_WIN // 2, tbody, 0)
    drain_sc(0, ss0)
    drain_sc(1, ss1)

    @pl.when(s < GRP_EXTRA)
    def _extra():
      g0 = base + GRP_BASE
      pltpu.sync_copy(idx_hbm.at[pl.ds(g0, 1)], idx_v.at[0, pl.ds(0, 1)])
      for fb in range(2):
        pltpu.sync_copy(
            edges_hbm.at[pl.ds(fb * (8 * NG) + g0 * 8, 8)],
            data_v.at[0, pl.ds(fb * 8 * W_GRPS, 8), pl.ds(0, GRP)])

      def blk16(k16, carry):
        e0 = k16 * 16
        for k in range(16):
          ev = jnp.full((16,), e0 + k, jnp.int32)
          row = plsc.load_gather(data_v.at[0], [frow, ev])
          rows_v[0, e0 + k] = row
        return carry

      lax.fori_loop(0, GRP // 16, blk16, 0)
      pltpu.sync_copy(rows_v.at[0, pl.ds(0, GRP)],
                      acc_sh.at[idx_v.at[0].at[0]], add=True)

    plsc.subcore_barrier()

    pltpu.sync_copy(acc_sh.at[pl.ds(s * ROWS_PER_TILE, ROWS_PER_TILE)],
                    rows_v.at[0, pl.ds(0, ROWS_PER_TILE)])
    pltpu.sync_copy(rows_v.at[0, pl.ds(0, ROWS_PER_TILE)],
                    out_hbm.at[c].at[pl.ds(s * ROWS_PER_TILE, ROWS_PER_TILE)])

  return seg_sum(edges2d, idx2d)


ROW_BLK = 1000


def _tc_mlp_ln(nodes, agg2, globals_, W1, b1, W2, b2, gamma, beta):
  grid = (N_NODES // ROW_BLK,)

  def body(nodes_ref, agg_ref, g_ref, w1_ref, b1_ref, w2_ref, b2_ref,
           gamma_ref, beta_ref, out_ref):
    agg = agg_ref[0] + agg_ref[1]
    w1n = w1_ref[:D_FEAT]
    w1g = w1_ref[D_FEAT:D_FEAT + D_GLOBAL]
    w1f = w1_ref[D_FEAT + D_GLOBAL:]
    bias1 = b1_ref[...] + jnp.dot(g_ref[...], w1g,
                                  preferred_element_type=jnp.float32)
    x = (jnp.dot(nodes_ref[...], w1n, preferred_element_type=jnp.float32)
         + jnp.dot(agg, w1f, preferred_element_type=jnp.float32)
         + bias1)
    h = jnp.maximum(x, 0.0)
    h = jnp.dot(h, w2_ref[...], preferred_element_type=jnp.float32)
    h = jnp.maximum(h + b2_ref[...], 0.0)
    mean = jnp.mean(h, axis=1, keepdims=True)
    d = h - mean
    var = jnp.mean(d * d, axis=1, keepdims=True)
    out_ref[...] = gamma_ref[...] * d * lax.rsqrt(var + 1e-3) + beta_ref[...]

  return pl.pallas_call(
      body,
      grid=grid,
      in_specs=[
          pl.BlockSpec((ROW_BLK, D_FEAT), lambda i: (i, 0)),
          pl.BlockSpec((NC, ROW_BLK, D_EDGE), lambda i: (0, i, 0)),
          pl.BlockSpec((1, D_GLOBAL), lambda i: (0, 0)),
          pl.BlockSpec((D_FEAT + D_GLOBAL + D_EDGE, H_DIM), lambda i: (0, 0)),
          pl.BlockSpec((1, H_DIM), lambda i: (0, 0)),
          pl.BlockSpec((H_DIM, H_DIM), lambda i: (0, 0)),
          pl.BlockSpec((1, H_DIM), lambda i: (0, 0)),
          pl.BlockSpec((1, H_DIM), lambda i: (0, 0)),
          pl.BlockSpec((1, H_DIM), lambda i: (0, 0)),
      ],
      out_specs=pl.BlockSpec((ROW_BLK, H_DIM), lambda i: (i, 0)),
      out_shape=jax.ShapeDtypeStruct((N_NODES, H_DIM), jnp.float32),
      compiler_params=pltpu.CompilerParams(
          dimension_semantics=("arbitrary",),
      ),
  )(nodes, agg2, globals_, W1, b1, W2, b2, gamma, beta)


@jax.jit
def kernel(nodes, globals_, n_node, hyperedges, hyperedge_index,
           W1, b1, W2, b2, gamma, beta):
  del n_node
  idx2d = hyperedge_index.reshape(NG, GRP)
  edges2d = (hyperedges.T.reshape(2, 8, NG, GRP)
             .transpose(0, 2, 1, 3).reshape(2 * NG * 8, GRP))
  agg2 = _sc_segment_sum(edges2d, idx2d)
  return _tc_mlp_ln(
      nodes, agg2, globals_, W1,
      b1.reshape(1, H_DIM), W2, b2.reshape(1, H_DIM),
      gamma.reshape(1, H_DIM), beta.reshape(1, H_DIM),
  )


# speedup vs baseline: 2.7515x; 1.5241x over previous
import functools

import jax
import jax.numpy as jnp
from jax import lax
from jax.experimental import pallas as pl
from jax.experimental.pallas import tpu as pltpu
from jax.experimental.pallas import tpu_sc as plsc

N_NODES = 10000
N_EDGES = 320000
D_EDGE = 16
D_FEAT = 128
D_GLOBAL = 32
H_DIM = 128

GRP = 128
NG = N_EDGES // GRP
NC = 2
NS = 16
GRP_PER_SC = NG // NC
GRP_BASE = GRP_PER_SC // NS
GRP_EXTRA = GRP_PER_SC - GRP_BASE * NS
W_GRPS = 13
N_WIN = GRP_BASE // W_GRPS
ROWS_PER_TILE = N_NODES // NS


WIN_ROWS = 2 * W_GRPS * 8
PAD = GRP + 1


def _sc_segment_sum(edges2d, idx2d):
  mesh = plsc.VectorSubcoreMesh(core_axis_name="c", subcore_axis_name="s")

  @functools.partial(
      pl.kernel,
      out_type=jax.ShapeDtypeStruct((NC, N_NODES, D_EDGE), jnp.float32),
      mesh=mesh,
      scratch_types=[
          pltpu.VMEM((N_WIN, W_GRPS, GRP), jnp.int32),
          pltpu.VMEM((2, WIN_ROWS, PAD), jnp.float32),
          pltpu.VMEM((2, W_GRPS * GRP, D_EDGE), jnp.float32),
          pltpu.VMEM_SHARED((N_NODES, D_EDGE), jnp.float32),
          pltpu.SemaphoreType.DMA,
          pltpu.SemaphoreType.DMA,
          pltpu.SemaphoreType.DMA,
          pltpu.SemaphoreType.DMA,
      ],
      compiler_params=pltpu.CompilerParams(use_tc_tiling_on_sc=False,
                                           needs_layout_passes=False),
  )
  def seg_sum(edges_hbm, idx_hbm, out_hbm, idx_v, data_v, rows_v, acc_sh,
              si0, si1, ss0, ss1):
    c = lax.axis_index("c")
    s = lax.axis_index("s")

    zrow = jnp.zeros((D_EDGE,), jnp.float32)

    def zero_body(i, carry):
      rows_v[0, i] = zrow
      return carry

    lax.fori_loop(0, ROWS_PER_TILE, zero_body, 0)
    pltpu.sync_copy(rows_v.at[0, pl.ds(0, ROWS_PER_TILE)],
                    acc_sh.at[pl.ds(s * ROWS_PER_TILE, ROWS_PER_TILE)])
    plsc.subcore_barrier()

    base = c * GRP_PER_SC + s * GRP_BASE + jnp.minimum(s, GRP_EXTRA)

    lane = lax.iota(jnp.int32, 16)
    frow = (lane >> 3) * (8 * W_GRPS) + (lane & 7)

    def start_in(w, b, sem):
      g0 = base + w * W_GRPS
      pltpu.async_copy(idx_hbm.at[pl.ds(g0, W_GRPS)], idx_v.at[w], sem)
      for fb in range(2):
        pltpu.async_copy(
            edges_hbm.at[pl.ds(fb * (8 * NG) + g0 * 8, 8 * W_GRPS)],
            data_v.at[b, pl.ds(fb * 8 * W_GRPS, 8 * W_GRPS),
                      pl.ds(0, GRP)], sem)

    def drain_in(b, sem):
      del b
      pltpu.make_async_copy(idx_hbm.at[pl.ds(0, W_GRPS)], idx_v.at[0],
                            sem).wait()
      pltpu.make_async_copy(out_hbm.at[0].at[pl.ds(0, W_GRPS * GRP)],
                            rows_v.at[0], sem).wait()

    def drain_sc(b, sem):
      pltpu.make_async_copy(out_hbm.at[0].at[pl.ds(0, W_GRPS * GRP)],
                            rows_v.at[b], sem).wait()

    def transpose_scatter(w, b, sem):
      def grp_body(g, carry):
        rowg = frow + g * 8

        @plsc.parallel_loop(0, GRP, 1, unroll=16)
        def _edges(e):
          ev = jnp.full((16,), e, jnp.int32)
          row = plsc.load_gather(data_v.at[b], [rowg, ev])
          rows_v[b, g * GRP + e] = row

        pltpu.async_copy(rows_v.at[b, pl.ds(g * GRP, GRP)],
                         acc_sh.at[idx_v.at[w].at[g]], sem, add=True)
        return carry

      lax.fori_loop(0, W_GRPS, grp_body, 0)

    start_in(0, 0, si0)
    start_in(1, 1, si1)

    def tbody(t, carry):
      w0 = 2 * t
      drain_in(0, si0)

      @pl.when(t >= 1)
      def _():
        drain_sc(0, ss0)

      transpose_scatter(w0, 0, ss0)

      @pl.when(w0 + 2 < N_WIN)
      def _():
        start_in(w0 + 2, 0, si0)

      drain_in(1, si1)

      @pl.when(t >= 1)
      def _():
        drain_sc(1, ss1)

      transpose_scatter(w0 + 1, 1, ss1)

      @pl.when(w0 + 3 < N_WIN)
      def _():
        start_in(w0 + 3, 1, si1)

      return carry

    lax.fori_loop(0, N_WIN // 2, tbody, 0)
    drain_sc(0, ss0)
    drain_sc(1, ss1)

    @pl.when(s < GRP_EXTRA)
    def _extra():
      g0 = base + GRP_BASE
      pltpu.sync_copy(idx_hbm.at[pl.ds(g0, 1)], idx_v.at[0, pl.ds(0, 1)])
      for fb in range(2):
        pltpu.sync_copy(
            edges_hbm.at[pl.ds(fb * (8 * NG) + g0 * 8, 8)],
            data_v.at[0, pl.ds(fb * 8 * W_GRPS, 8), pl.ds(0, GRP)])

      @plsc.parallel_loop(0, GRP, 1, unroll=16)
      def _edges(e):
        ev = jnp.full((16,), e, jnp.int32)
        row = plsc.load_gather(data_v.at[0], [frow, ev])
        rows_v[0, e] = row

      pltpu.sync_copy(rows_v.at[0, pl.ds(0, GRP)],
                      acc_sh.at[idx_v.at[0].at[0]], add=True)

    plsc.subcore_barrier()

    pltpu.sync_copy(acc_sh.at[pl.ds(s * ROWS_PER_TILE, ROWS_PER_TILE)],
                    rows_v.at[0, pl.ds(0, ROWS_PER_TILE)])
    pltpu.sync_copy(rows_v.at[0, pl.ds(0, ROWS_PER_TILE)],
                    out_hbm.at[c].at[pl.ds(s * ROWS_PER_TILE, ROWS_PER_TILE)])

  return seg_sum(edges2d, idx2d)


ROW_BLK = 1000


def _tc_mlp_ln(nodes, agg2, globals_, W1, b1, W2, b2, gamma, beta):
  grid = (N_NODES // ROW_BLK,)

  def body(nodes_ref, agg_ref, g_ref, w1_ref, b1_ref, w2_ref, b2_ref,
           gamma_ref, beta_ref, out_ref):
    agg = agg_ref[0] + agg_ref[1]
    w1n = w1_ref[:D_FEAT]
    w1g = w1_ref[D_FEAT:D_FEAT + D_GLOBAL]
    w1f = w1_ref[D_FEAT + D_GLOBAL:]
    bias1 = b1_ref[...] + jnp.dot(g_ref[...], w1g,
                                  preferred_element_type=jnp.float32)
    x = (jnp.dot(nodes_ref[...], w1n, preferred_element_type=jnp.float32)
         + jnp.dot(agg, w1f, preferred_element_type=jnp.float32)
         + bias1)
    h = jnp.maximum(x, 0.0)
    h = jnp.dot(h, w2_ref[...], preferred_element_type=jnp.float32)
    h = jnp.maximum(h + b2_ref[...], 0.0)
    mean = jnp.mean(h, axis=1, keepdims=True)
    d = h - mean
    var = jnp.mean(d * d, axis=1, keepdims=True)
    out_ref[...] = gamma_ref[...] * d * lax.rsqrt(var + 1e-3) + beta_ref[...]

  return pl.pallas_call(
      body,
      grid=grid,
      in_specs=[
          pl.BlockSpec((ROW_BLK, D_FEAT), lambda i: (i, 0)),
          pl.BlockSpec((NC, ROW_BLK, D_EDGE), lambda i: (0, i, 0)),
          pl.BlockSpec((1, D_GLOBAL), lambda i: (0, 0)),
          pl.BlockSpec((D_FEAT + D_GLOBAL + D_EDGE, H_DIM), lambda i: (0, 0)),
          pl.BlockSpec((1, H_DIM), lambda i: (0, 0)),
          pl.BlockSpec((H_DIM, H_DIM), lambda i: (0, 0)),
          pl.BlockSpec((1, H_DIM), lambda i: (0, 0)),
          pl.BlockSpec((1, H_DIM), lambda i: (0, 0)),
          pl.BlockSpec((1, H_DIM), lambda i: (0, 0)),
      ],
      out_specs=pl.BlockSpec((ROW_BLK, H_DIM), lambda i: (i, 0)),
      out_shape=jax.ShapeDtypeStruct((N_NODES, H_DIM), jnp.float32),
      compiler_params=pltpu.CompilerParams(
          dimension_semantics=("arbitrary",),
      ),
  )(nodes, agg2, globals_, W1, b1, W2, b2, gamma, beta)


@jax.jit
def kernel(nodes, globals_, n_node, hyperedges, hyperedge_index,
           W1, b1, W2, b2, gamma, beta):
  del n_node
  idx2d = hyperedge_index.reshape(NG, GRP)
  edges2d = (hyperedges.T.reshape(2, 8, NG, GRP)
             .transpose(0, 2, 1, 3).reshape(2 * NG * 8, GRP))
  agg2 = _sc_segment_sum(edges2d, idx2d)
  return _tc_mlp_ln(
      nodes, agg2, globals_, W1,
      b1.reshape(1, H_DIM), W2, b2.reshape(1, H_DIM),
      gamma.reshape(1, H_DIM), beta.reshape(1, H_DIM),
  )


# speedup vs baseline: 3.0799x; 1.1193x over previous
import functools

import jax
import jax.numpy as jnp
from jax import lax
from jax.experimental import pallas as pl
from jax.experimental.pallas import tpu as pltpu
from jax.experimental.pallas import tpu_sc as plsc

N_NODES = 10000
N_EDGES = 320000
D_EDGE = 16
D_FEAT = 128
D_GLOBAL = 32
H_DIM = 128

GRP = 128
NG = N_EDGES // GRP
NC = 2
NS = 16
GRP_PER_SC = NG // NC
GRP_BASE = GRP_PER_SC // NS
GRP_EXTRA = GRP_PER_SC - GRP_BASE * NS
W_GRPS = 13
N_WIN = GRP_BASE // W_GRPS
ROWS_PER_TILE = N_NODES // NS


WIN_ROWS = 2 * W_GRPS * 8
PAD = GRP + 1
N_TC = 79
N_PAD = N_TC * GRP
ZROWS_PER_TILE = N_PAD // NS


def _sc_segment_sum(edges2d, idx2d):
  mesh = plsc.VectorSubcoreMesh(core_axis_name="c", subcore_axis_name="s")

  @functools.partial(
      pl.kernel,
      out_type=[
          jax.ShapeDtypeStruct((NC, 2, N_TC, 8, GRP), jnp.float32),
          jax.ShapeDtypeStruct((W_GRPS * GRP, D_EDGE), jnp.float32),
      ],
      mesh=mesh,
      scratch_types=[
          pltpu.VMEM((4, W_GRPS, GRP), jnp.int32),
          pltpu.VMEM((2, WIN_ROWS, PAD), jnp.float32),
          pltpu.VMEM((2, W_GRPS * GRP, D_EDGE), jnp.float32),
          pltpu.VMEM((GRP, D_EDGE + 1), jnp.float32),
          pltpu.VMEM((8, GRP), jnp.float32),
          pltpu.VMEM_SHARED((N_PAD, D_EDGE), jnp.float32),
          pltpu.SemaphoreType.DMA,
          pltpu.SemaphoreType.DMA,
          pltpu.SemaphoreType.DMA,
          pltpu.SemaphoreType.DMA,
      ],
      compiler_params=pltpu.CompilerParams(use_tc_tiling_on_sc=False,
                                           needs_layout_passes=False),
  )
  def seg_sum(edges_hbm, idx_hbm, out_hbm, dummy_hbm, idx_v, data_v, rows_v,
              stage_in, stage_out, acc_sh, si0, si1, ss0, ss1):
    c = lax.axis_index("c")
    s = lax.axis_index("s")

    zrow = jnp.zeros((D_EDGE,), jnp.float32)

    def zero_body(i, carry):
      rows_v[0, i] = zrow
      return carry

    lax.fori_loop(0, ZROWS_PER_TILE, zero_body, 0)
    pltpu.sync_copy(rows_v.at[0, pl.ds(0, ZROWS_PER_TILE)],
                    acc_sh.at[pl.ds(s * ZROWS_PER_TILE, ZROWS_PER_TILE)])
    plsc.subcore_barrier()

    base = c * GRP_PER_SC + s * GRP_BASE + jnp.minimum(s, GRP_EXTRA)

    lane = lax.iota(jnp.int32, 16)
    frow = (lane >> 3) * (8 * W_GRPS) + (lane & 7)

    def start_in(w, b, sem):
      g0 = base + w * W_GRPS
      pltpu.async_copy(idx_hbm.at[pl.ds(g0, W_GRPS)], idx_v.at[w & 3], sem)
      for fb in range(2):
        pltpu.async_copy(
            edges_hbm.at[pl.ds(fb * (8 * NG) + g0 * 8, 8 * W_GRPS)],
            data_v.at[b, pl.ds(fb * 8 * W_GRPS, 8 * W_GRPS),
                      pl.ds(0, GRP)], sem)

    def drain_in(b, sem):
      del b
      pltpu.make_async_copy(idx_hbm.at[pl.ds(0, W_GRPS)], idx_v.at[0],
                            sem).wait()
      pltpu.make_async_copy(dummy_hbm, rows_v.at[0], sem).wait()

    def drain_sc(b, sem):
      pltpu.make_async_copy(dummy_hbm, rows_v.at[b], sem).wait()

    def transpose_scatter(w, b, sem):
      def grp_body(g, carry):
        rowg = frow + g * 8

        @plsc.parallel_loop(0, GRP, 1, unroll=16)
        def _edges(e):
          ev = jnp.full((16,), e, jnp.int32)
          row = plsc.load_gather(data_v.at[b], [rowg, ev])
          rows_v[b, g * GRP + e] = row

        pltpu.async_copy(rows_v.at[b, pl.ds(g * GRP, GRP)],
                         acc_sh.at[idx_v.at[w & 3].at[g]], sem, add=True)
        return carry

      lax.fori_loop(0, W_GRPS, grp_body, 0)

    start_in(0, 0, si0)
    start_in(1, 1, si1)

    def tbody(t, carry):
      w0 = 2 * t
      drain_in(0, si0)

      @pl.when(t >= 1)
      def _():
        drain_sc(0, ss0)

      transpose_scatter(w0, 0, ss0)

      @pl.when(w0 + 2 < N_WIN)
      def _():
        start_in(w0 + 2, 0, si0)

      drain_in(1, si1)

      @pl.when(t >= 1)
      def _():
        drain_sc(1, ss1)

      transpose_scatter(w0 + 1, 1, ss1)

      @pl.when(w0 + 3 < N_WIN)
      def _():
        start_in(w0 + 3, 1, si1)

      return carry

    lax.fori_loop(0, N_WIN // 2, tbody, 0)
    drain_sc(0, ss0)
    drain_sc(1, ss1)

    @pl.when(s < GRP_EXTRA)
    def _extra():
      g0 = base + GRP_BASE
      pltpu.sync_copy(idx_hbm.at[pl.ds(g0, 1)], idx_v.at[0, pl.ds(0, 1)])
      for fb in range(2):
        pltpu.sync_copy(
            edges_hbm.at[pl.ds(fb * (8 * NG) + g0 * 8, 8)],
            data_v.at[0, pl.ds(fb * 8 * W_GRPS, 8), pl.ds(0, GRP)])

      @plsc.parallel_loop(0, GRP, 1, unroll=16)
      def _edges(e):
        ev = jnp.full((16,), e, jnp.int32)
        row = plsc.load_gather(data_v.at[0], [frow, ev])
        rows_v[0, e] = row

      pltpu.sync_copy(rows_v.at[0, pl.ds(0, GRP)],
                      acc_sh.at[idx_v.at[0].at[0]], add=True)

    plsc.subcore_barrier()

    lane16 = lax.iota(jnp.int32, 16)
    n_tc = jnp.where(s < NS - 1, 5, 4)

    def out_block(j, carry):
      tc = s * 5 + j
      pltpu.sync_copy(acc_sh.at[pl.ds(tc * GRP, GRP)],
                      stage_in.at[:, pl.ds(0, D_EDGE)])
      for fb in range(2):

        @plsc.parallel_loop(0, 64, 1, unroll=8)
        def _t(m, fb=fb):
          fi = m >> 3
          e0 = (m & 7) << 4
          v = plsc.load_gather(
              stage_in,
              [e0 + lane16, jnp.full((16,), fb * 8 + fi, jnp.int32)])
          stage_out[fi, pl.ds(e0, 16)] = v

        pltpu.sync_copy(stage_out, out_hbm.at[c, fb, tc])
      return carry

    lax.fori_loop(0, n_tc, out_block, 0)

  return seg_sum(edges2d, idx2d)


ROW_BLK = 2048


def _tc_mlp_ln(nodes, agg_t, globals_, W1, b1, W2, b2, gamma, beta):
  grid = (pl.cdiv(N_NODES, ROW_BLK),)

  def body(nodes_ref, agg_ref, g_ref, w1_ref, b1_ref, w2_ref, b2_ref,
           gamma_ref, beta_ref, out_ref):
    agg = agg_ref[0] + agg_ref[1]
    w1n = w1_ref[:D_FEAT]
    w1g = w1_ref[D_FEAT:D_FEAT + D_GLOBAL]
    w1f = w1_ref[D_FEAT + D_GLOBAL:]
    bias1 = b1_ref[...] + jnp.dot(g_ref[...], w1g,
                                  preferred_element_type=jnp.float32)
    xagg = lax.dot_general(agg, w1f, (((0,), (0,)), ((), ())),
                           preferred_element_type=jnp.float32)
    x = (jnp.dot(nodes_ref[...], w1n, preferred_element_type=jnp.float32)
         + xagg + bias1)
    h = jnp.maximum(x, 0.0)
    h = jnp.dot(h, w2_ref[...], preferred_element_type=jnp.float32)
    h = jnp.maximum(h + b2_ref[...], 0.0)
    mean = jnp.mean(h, axis=1, keepdims=True)
    d = h - mean
    var = jnp.mean(d * d, axis=1, keepdims=True)
    out_ref[...] = gamma_ref[...] * d * lax.rsqrt(var + 1e-3) + beta_ref[...]

  return pl.pallas_call(
      body,
      grid=grid,
      in_specs=[
          pl.BlockSpec((ROW_BLK, D_FEAT), lambda i: (i, 0)),
          pl.BlockSpec((NC, D_EDGE, ROW_BLK), lambda i: (0, 0, i)),
          pl.BlockSpec((1, D_GLOBAL), lambda i: (0, 0)),
          pl.BlockSpec((D_FEAT + D_GLOBAL + D_EDGE, H_DIM), lambda i: (0, 0)),
          pl.BlockSpec((1, H_DIM), lambda i: (0, 0)),
          pl.BlockSpec((H_DIM, H_DIM), lambda i: (0, 0)),
          pl.BlockSpec((1, H_DIM), lambda i: (0, 0)),
          pl.BlockSpec((1, H_DIM), lambda i: (0, 0)),
          pl.BlockSpec((1, H_DIM), lambda i: (0, 0)),
      ],
      out_specs=pl.BlockSpec((ROW_BLK, H_DIM), lambda i: (i, 0)),
      out_shape=jax.ShapeDtypeStruct((N_NODES, H_DIM), jnp.float32),
      compiler_params=pltpu.CompilerParams(
          dimension_semantics=("arbitrary",),
      ),
  )(nodes, agg_t, globals_, W1, b1, W2, b2, gamma, beta)


@jax.jit
def kernel(nodes, globals_, n_node, hyperedges, hyperedge_index,
           W1, b1, W2, b2, gamma, beta):
  del n_node
  idx2d = hyperedge_index.reshape(NG, GRP)
  edges2d = (hyperedges.T.reshape(2, 8, NG, GRP)
             .transpose(0, 2, 1, 3).reshape(2 * NG * 8, GRP))
  agg5, _unused = _sc_segment_sum(edges2d, idx2d)
  agg_t = agg5.transpose(0, 1, 3, 2, 4).reshape(NC, D_EDGE, N_TC * GRP)
  return _tc_mlp_ln(
      nodes, agg_t, globals_, W1,
      b1.reshape(1, H_DIM), W2, b2.reshape(1, H_DIM),
      gamma.reshape(1, H_DIM), beta.reshape(1, H_DIM),
  )
